# Initial kernel scaffold; baseline (speedup 1.0000x reference)
import jax
import jax.numpy as jnp
from jax.experimental import pallas as pl


def kernel(node_inputs, node_indices, neighbour_indices, temp_node_indices, temp_neighbour_indices, embed, mW0, mb0, mW1, mb1, mW2, mb2, giW, giU, gib, gtW, gtU, gtb, rW0, rb0, rW1, rb1, rW2, rb2):
    raise NotImplementedError("write your pallas kernel here")


# speedup vs baseline: 1.5587x; 1.5587x over previous
import jax
import jax.numpy as jnp
from jax.experimental import pallas as pl

B = 4
N_NODES = 10000
WINDOW = 4
D = 128
H = 64
N_PARTS = 10
E = 320000
ET = 120000
T = 2
TOTAL = B * WINDOW * N_NODES
_R = 1000


def _edge_body(gi_ref, gj_ref, mb0_ref, mW1_ref, mb1_ref, mW2_ref, mb2_ref,
               m1_ref, m2_ref):
    gi = gi_ref[...]
    gj = gj_ref[...]
    mb0 = mb0_ref[...]
    W1 = mW1_ref[...]
    b1 = mb1_ref[...]
    W2 = mW2_ref[...]
    b2 = mb2_ref[...]
    x1 = gi[:, :H] + gj[:, H:] + mb0
    x2 = gj[:, :H] + gi[:, H:] + mb0
    for x, out in ((x1, m1_ref), (x2, m2_ref)):
        h = jax.nn.gelu(x)
        h = jax.nn.gelu(jnp.dot(h, W1, preferred_element_type=jnp.float32) + b1)
        out[...] = jnp.dot(h, W2, preferred_element_type=jnp.float32) + b2


def _edge_mlp(gi, gj, mb0, mW1, mb1, mW2, mb2):
    rows = gi.shape[0]
    full = lambda s: pl.BlockSpec(s, lambda i: (0,) * len(s))
    return pl.pallas_call(
        _edge_body,
        grid=(rows // _R,),
        in_specs=[
            pl.BlockSpec((_R, D), lambda i: (i, 0)),
            pl.BlockSpec((_R, D), lambda i: (i, 0)),
            full((1, H)), full((H, H)), full((1, H)), full((H, D)), full((1, D)),
        ],
        out_specs=[pl.BlockSpec((_R, D), lambda i: (i, 0))] * 2,
        out_shape=[jax.ShapeDtypeStruct((rows, D), jnp.float32)] * 2,
    )(gi, gj, mb0, mW1, mb1, mW2, mb2)


def _gru_body(s_ref, inv_ref, h_ref, W_ref, U_ref, b_ref, W0_ref,
              ns_ref, ab_ref):
    x = s_ref[...] * inv_ref[...]
    h = h_ref[...]
    gx = jnp.dot(x, W_ref[...], preferred_element_type=jnp.float32) + b_ref[...]
    gh = jnp.dot(h, U_ref[...], preferred_element_type=jnp.float32)
    z = jax.nn.sigmoid(gx[:, :D] + gh[:, :D])
    r = jax.nn.sigmoid(gx[:, D:2 * D] + gh[:, D:2 * D])
    n = jnp.tanh(gx[:, 2 * D:] + r * gh[:, 2 * D:])
    ns = z * h + (1.0 - z) * n
    ns_ref[...] = ns
    ab_ref[...] = jnp.dot(ns, W0_ref[...], preferred_element_type=jnp.float32)


def _gru_update(sums, inv, ns, W, U, b, W0cat):
    full = lambda s: pl.BlockSpec(s, lambda i: (0,) * len(s))
    return pl.pallas_call(
        _gru_body,
        grid=(TOTAL // _R,),
        in_specs=[
            pl.BlockSpec((_R, D), lambda i: (i, 0)),
            pl.BlockSpec((_R, 1), lambda i: (i, 0)),
            pl.BlockSpec((_R, D), lambda i: (i, 0)),
            full((D, 3 * D)), full((D, 3 * D)), full((1, 3 * D)), full((D, D)),
        ],
        out_specs=[pl.BlockSpec((_R, D), lambda i: (i, 0))] * 2,
        out_shape=[jax.ShapeDtypeStruct((TOTAL, D), jnp.float32)] * 2,
    )(sums, inv, ns, W, U, b, W0cat)


def _mm_body(x_ref, w_ref, o_ref):
    o_ref[...] = jnp.dot(x_ref[...], w_ref[...],
                         preferred_element_type=jnp.float32)


def _matmul(x, w):
    return pl.pallas_call(
        _mm_body,
        grid=(x.shape[0] // _R,),
        in_specs=[
            pl.BlockSpec((_R, x.shape[1]), lambda i: (i, 0)),
            pl.BlockSpec(w.shape, lambda i: (0, 0)),
        ],
        out_specs=pl.BlockSpec((_R, w.shape[1]), lambda i: (i, 0)),
        out_shape=jax.ShapeDtypeStruct((x.shape[0], w.shape[1]), jnp.float32),
    )(x, w)


def _read_body(x_ref, W0_ref, b0_ref, W1_ref, b1_ref, W2_ref, b2_ref, o_ref):
    h = jax.nn.gelu(jnp.dot(x_ref[...], W0_ref[...],
                            preferred_element_type=jnp.float32) + b0_ref[...])
    h = jax.nn.gelu(jnp.dot(h, W1_ref[...],
                            preferred_element_type=jnp.float32) + b1_ref[...])
    o_ref[...] = jnp.dot(h, W2_ref[...],
                         preferred_element_type=jnp.float32) + b2_ref[...]


def _readout(x, W0, b0, W1, b1, W2, b2):
    full = lambda s: pl.BlockSpec(s, lambda i: (0,) * len(s))
    return pl.pallas_call(
        _read_body,
        grid=(x.shape[0] // _R,),
        in_specs=[
            pl.BlockSpec((_R, D), lambda i: (i, 0)),
            full((D, H)), full((1, H)), full((H, H)), full((1, H)),
            full((H, N_PARTS)), full((1, N_PARTS)),
        ],
        out_specs=pl.BlockSpec((_R, N_PARTS), lambda i: (i, 0)),
        out_shape=jax.ShapeDtypeStruct((x.shape[0], N_PARTS), jnp.float32),
    )(x, W0, b0, W1, b1, W2, b2)


def _phase(ns, AB, idx_i, idx_j, inv, mb0, mW1, mb1, mW2, mb2, W, U, b, W0cat):
    gi = jnp.take(AB, idx_i, axis=0)
    gj = jnp.take(AB, idx_j, axis=0)
    m1, m2 = _edge_mlp(gi, gj, mb0, mW1, mb1, mW2, mb2)
    sums = (jax.ops.segment_sum(m1, idx_j, num_segments=TOTAL) +
            jax.ops.segment_sum(m2, idx_i, num_segments=TOTAL))
    return _gru_update(sums, inv, ns, W, U, b, W0cat)


def kernel(node_inputs, node_indices, neighbour_indices, temp_node_indices,
           temp_neighbour_indices, embed, mW0, mb0, mW1, mb1, mW2, mb2,
           giW, giU, gib, gtW, gtU, gtb, rW0, rb0, rW1, rb1, rW2, rb2):
    e = jnp.take(embed, node_inputs, axis=0)
    ns = jnp.broadcast_to(e[:, None], (B, WINDOW, N_NODES, D)).reshape(TOTAL, D)

    W0cat = jnp.concatenate([mW0[:D], mW0[D:]], axis=1)
    mb0_2 = mb0[None]
    mb1_2 = mb1[None]
    mb2_2 = mb2[None]
    gib_2 = gib[None]
    gtb_2 = gtb[None]

    ones_e = jnp.ones((E,), jnp.float32)
    ones_t = jnp.ones((ET,), jnp.float32)
    cnt_int = (jax.ops.segment_sum(ones_e, neighbour_indices, num_segments=TOTAL) +
               jax.ops.segment_sum(ones_e, node_indices, num_segments=TOTAL))
    cnt_tmp = (jax.ops.segment_sum(ones_t, temp_neighbour_indices, num_segments=TOTAL) +
               jax.ops.segment_sum(ones_t, temp_node_indices, num_segments=TOTAL))
    inv_int = (1.0 / jnp.maximum(cnt_int, 1.0))[:, None]
    inv_tmp = (1.0 / jnp.maximum(cnt_tmp, 1.0))[:, None]

    AB = _matmul(ns, W0cat)
    for _ in range(T):
        ns, AB = _phase(ns, AB, node_indices, neighbour_indices, inv_int,
                        mb0_2, mW1, mb1_2, mW2, mb2_2, giW, giU, gib_2, W0cat)
        ns, AB = _phase(ns, AB, temp_node_indices, temp_neighbour_indices,
                        inv_tmp, mb0_2, mW1, mb1_2, mW2, mb2_2,
                        gtW, gtU, gtb_2, W0cat)

    feats = ns.reshape(B, WINDOW, N_NODES, D)[:, 0].reshape(B * N_NODES, D)
    out = _readout(feats, rW0, rb0[None], rW1, rb1[None], rW2, rb2[None])
    return out.reshape(B, N_NODES, N_PARTS)
